# Initial kernel scaffold; baseline (speedup 1.0000x reference)
import jax
import jax.numpy as jnp
from jax.experimental import pallas as pl


def kernel(nh, eh, edge_index, W1, b1, W2, b2):
    raise NotImplementedError("write your pallas kernel here")


# speedup vs baseline: 5.6251x; 5.6251x over previous
import jax
import jax.numpy as jnp
from jax import lax
from jax.experimental import pallas as pl
from jax.experimental.pallas import tpu as pltpu
from jax.experimental.pallas import tpu_sc as plsc

N = 10000
E = 320000
D = 128

_CHUNK = 128
_NCHUNKS = E // _CHUNK
_NC = 2
_NS = 16
_NW = _NC * _NS
_FULL_ITERS = _NCHUNKS // _NW
_EXTRA = _NCHUNKS - _FULL_ITERS * _NW
_NP = 10112
_ROWS_PER_TILE = _NP // _NS
_CW = 16


def _worker_ids():
    cid = lax.axis_index("c")
    sid = lax.axis_index("s")
    return cid, sid, cid * _NS + sid


def _for_my_chunks(wid, fn):
    def _loop(i, carry):
        fn(wid + i * _NW)
        return carry
    lax.fori_loop(0, _FULL_ITERS, _loop, 0)

    @pl.when(wid < _EXTRA)
    def _():
        fn(_FULL_ITERS * _NW + wid)


def _sc_sums_body(nh_hbm, src_hbm, dst_hbm, sums_out,
                  idx_v, dst_v, rows_v, acc, sem):
    cid, sid, wid = _worker_ids()
    base0 = sid * _ROWS_PER_TILE

    def _init(r, carry):
        for cc in range(D // 16):
            rows_v[r, pl.ds(cc * 16, 16)] = jnp.zeros((16,), jnp.float32)
        return carry
    lax.fori_loop(0, _CHUNK, _init, 0)
    off = 0
    while off < _ROWS_PER_TILE:
        sz = min(_CHUNK, _ROWS_PER_TILE - off)
        pltpu.sync_copy(rows_v.at[pl.ds(0, sz)], acc.at[pl.ds(base0 + off, sz)])
        off += sz
    plsc.subcore_barrier()

    def _do(chunk_id):
        base = chunk_id * _CHUNK
        pltpu.sync_copy(src_hbm.at[pl.ds(base, _CHUNK)], idx_v)
        pltpu.sync_copy(dst_hbm.at[pl.ds(base, _CHUNK)], dst_v)
        pltpu.async_copy(nh_hbm.at[idx_v], rows_v, sem).wait()
        pltpu.sync_copy(rows_v, acc.at[dst_v], add=True)
    _for_my_chunks(wid, _do)
    plsc.subcore_barrier()

    off = 0
    while off < _ROWS_PER_TILE:
        sz = min(_CHUNK, _ROWS_PER_TILE - off)
        row = base0 + off
        pltpu.sync_copy(acc.at[pl.ds(row, sz)], rows_v.at[pl.ds(0, sz)])
        pltpu.sync_copy(rows_v.at[pl.ds(0, sz)], sums_out.at[cid, pl.ds(row, sz)])
        off += sz


def _sc_cnt_body(dst_hbm, cnt_out, dst_v, ones_v, cnt_acc):
    cid, sid, wid = _worker_ids()
    base0 = sid * _ROWS_PER_TILE

    def _zero(r, carry):
        for cc in range(D // 16):
            ones_v[r, pl.ds(cc * 16, 16)] = jnp.zeros((16,), jnp.float32)
        return carry
    lax.fori_loop(0, _CHUNK, _zero, 0)
    off = 0
    while off < _ROWS_PER_TILE:
        sz = min(_CHUNK, _ROWS_PER_TILE - off)
        pltpu.sync_copy(ones_v.at[pl.ds(0, sz)], cnt_acc.at[pl.ds(base0 + off, sz)])
        off += sz

    def _fill1(r, carry):
        for cc in range(D // 16):
            ones_v[r, pl.ds(cc * 16, 16)] = jnp.ones((16,), jnp.float32)
        return carry
    lax.fori_loop(0, _CHUNK, _fill1, 0)
    plsc.subcore_barrier()

    def _do(chunk_id):
        pltpu.sync_copy(dst_hbm.at[pl.ds(chunk_id * _CHUNK, _CHUNK)], dst_v)
        pltpu.sync_copy(ones_v, cnt_acc.at[dst_v], add=True)
    _for_my_chunks(wid, _do)
    plsc.subcore_barrier()

    off = 0
    while off < _ROWS_PER_TILE:
        sz = min(_CHUNK, _ROWS_PER_TILE - off)
        row = base0 + off
        pltpu.sync_copy(cnt_acc.at[pl.ds(row, sz)], ones_v.at[pl.ds(0, sz)])
        pltpu.sync_copy(ones_v.at[pl.ds(0, sz)], cnt_out.at[cid, pl.ds(row, sz)])
        off += sz


@jax.jit
def _sc_scatter(nh, src, dst):
    mesh = plsc.VectorSubcoreMesh(core_axis_name="c", subcore_axis_name="s")
    sums = pl.kernel(
        _sc_sums_body,
        out_type=jax.ShapeDtypeStruct((_NC, _NP, D), jnp.float32),
        mesh=mesh,
        scratch_types=[
            pltpu.VMEM((_CHUNK,), jnp.int32),
            pltpu.VMEM((_CHUNK,), jnp.int32),
            pltpu.VMEM((_CHUNK, D), jnp.float32),
            pltpu.VMEM_SHARED((_NP, D), jnp.float32),
            pltpu.SemaphoreType.DMA,
        ],
    )(nh, src, dst)
    cnts = pl.kernel(
        _sc_cnt_body,
        out_type=jax.ShapeDtypeStruct((_NC, _NP, D), jnp.float32),
        mesh=mesh,
        scratch_types=[
            pltpu.VMEM((_CHUNK,), jnp.int32),
            pltpu.VMEM((_CHUNK, D), jnp.float32),
            pltpu.VMEM_SHARED((_NP, D), jnp.float32),
        ],
    )(dst)
    return sums, cnts


def _mlp_body(sums_ref, cnt_ref, nh_ref, w1a_ref, w1b_ref, b1_ref,
              w2_ref, b2_ref, out_ref):
    s = sums_ref[0] + sums_ref[1]
    c = cnt_ref[0] + cnt_ref[1]
    cnt = jnp.maximum(c[:, 0:1], 1.0)
    agg = s / cnt
    x = jnp.dot(nh_ref[...], w1a_ref[...], preferred_element_type=jnp.float32)
    x = x + jnp.dot(agg, w1b_ref[...], preferred_element_type=jnp.float32)
    h = jnp.maximum(x + b1_ref[...], 0.0)
    out_ref[...] = (jnp.dot(h, w2_ref[...], preferred_element_type=jnp.float32)
                    + b2_ref[...])


_BLK = 1000


@jax.jit
def _mlp(sums, cnts, nh, w1a, w1b, b1, w2, b2):
    grid = (N // _BLK,)
    return pl.pallas_call(
        _mlp_body,
        grid=grid,
        in_specs=[
            pl.BlockSpec((_NC, _BLK, D), lambda i: (0, i, 0)),
            pl.BlockSpec((_NC, _BLK, D), lambda i: (0, i, 0)),
            pl.BlockSpec((_BLK, D), lambda i: (i, 0)),
            pl.BlockSpec((D, D), lambda i: (0, 0)),
            pl.BlockSpec((D, D), lambda i: (0, 0)),
            pl.BlockSpec((1, D), lambda i: (0, 0)),
            pl.BlockSpec((D, D), lambda i: (0, 0)),
            pl.BlockSpec((1, D), lambda i: (0, 0)),
        ],
        out_specs=pl.BlockSpec((_BLK, D), lambda i: (i, 0)),
        out_shape=jax.ShapeDtypeStruct((N, D), jnp.float32),
    )(sums, cnts, nh, w1a, w1b, b1, w2, b2)


def kernel(nh, eh, edge_index, W1, b1, W2, b2):
    src = edge_index[0]
    dst = edge_index[1]
    sums, cnts = _sc_scatter(nh, src, dst)
    n_h = _mlp(sums, cnts, nh, W1[:D], W1[D:], b1.reshape(1, D),
               W2, b2.reshape(1, D))
    return (n_h, eh)


# speedup vs baseline: 6.6845x; 1.1883x over previous
import jax
import jax.numpy as jnp
from jax import lax
from jax.experimental import pallas as pl
from jax.experimental.pallas import tpu as pltpu
from jax.experimental.pallas import tpu_sc as plsc

N = 10000
E = 320000
D = 128

_NC = 2
_NS = 16
_NW = _NC * _NS
_NP = 10112
_ROWS_PER_TILE = _NP // _NS

_CS = 64
_NCH_S = E // _CS
_ITERS_S = _NCH_S // _NW
_BODIES_S = _ITERS_S // 2
_EXTRA_S = _NCH_S - _ITERS_S * _NW

_CC = 128
_NCH_C = E // _CC
_ITERS_C = _NCH_C // _NW
_BODIES_C = _ITERS_C // 2
_EXTRA_C = _NCH_C - _ITERS_C * _NW


def _worker_ids():
    cid = lax.axis_index("c")
    sid = lax.axis_index("s")
    return cid, sid, cid * _NS + sid


def _acc_slices(base0, piece):
    off = 0
    while off < _ROWS_PER_TILE:
        sz = min(piece, _ROWS_PER_TILE - off)
        yield base0 + off, sz
        off += sz


def _sc_sums_body(nh_hbm, src_hbm, dst_hbm, sums_out,
                  ia, ib, da, db, rows_a, rows_b, acc,
                  lsem, ssem, ga, gb):
    cid, sid, wid = _worker_ids()
    base0 = sid * _ROWS_PER_TILE

    def _init(r, carry):
        for cc in range(D // 16):
            rows_a[r, pl.ds(cc * 16, 16)] = jnp.zeros((16,), jnp.float32)
        return carry
    lax.fori_loop(0, _CS, _init, 0)
    for row, sz in _acc_slices(base0, _CS):
        pltpu.sync_copy(rows_a.at[pl.ds(0, sz)], acc.at[pl.ds(row, sz)])
    plsc.subcore_barrier()

    def _body(j, carry):
        c0 = (wid + (2 * j) * _NW) * _CS
        c1 = (wid + (2 * j + 1) * _NW) * _CS
        hs = (pltpu.async_copy(src_hbm.at[pl.ds(c0, _CS)], ia, lsem),
              pltpu.async_copy(dst_hbm.at[pl.ds(c0, _CS)], da, lsem),
              pltpu.async_copy(src_hbm.at[pl.ds(c1, _CS)], ib, lsem),
              pltpu.async_copy(dst_hbm.at[pl.ds(c1, _CS)], db, lsem))
        for h in hs:
            h.wait()
        gh0 = pltpu.async_copy(nh_hbm.at[ia], rows_a, ga)
        gh1 = pltpu.async_copy(nh_hbm.at[ib], rows_b, gb)
        gh0.wait()
        s0 = pltpu.async_copy(rows_a, acc.at[da], ssem, add=True)
        gh1.wait()
        s1 = pltpu.async_copy(rows_b, acc.at[db], ssem, add=True)
        s0.wait()
        s1.wait()
        return carry
    lax.fori_loop(0, _BODIES_S, _body, 0)

    @pl.when(wid < _EXTRA_S)
    def _():
        base = (_ITERS_S * _NW + wid) * _CS
        pltpu.sync_copy(src_hbm.at[pl.ds(base, _CS)], ia)
        pltpu.sync_copy(dst_hbm.at[pl.ds(base, _CS)], da)
        pltpu.async_copy(nh_hbm.at[ia], rows_a, ga).wait()
        pltpu.sync_copy(rows_a, acc.at[da], add=True)

    plsc.subcore_barrier()

    for row, sz in _acc_slices(base0, _CS):
        pltpu.sync_copy(acc.at[pl.ds(row, sz)], rows_a.at[pl.ds(0, sz)])
        pltpu.sync_copy(rows_a.at[pl.ds(0, sz)], sums_out.at[cid, pl.ds(row, sz)])


def _sc_cnt_body(dst_hbm, cnt_out, da, db, ones_v, acc, lsem, ssem):
    cid, sid, wid = _worker_ids()
    base0 = sid * _ROWS_PER_TILE

    def _zero(r, carry):
        for cc in range(D // 16):
            ones_v[r, pl.ds(cc * 16, 16)] = jnp.zeros((16,), jnp.float32)
        return carry
    lax.fori_loop(0, _CC, _zero, 0)
    for row, sz in _acc_slices(base0, _CC):
        pltpu.sync_copy(ones_v.at[pl.ds(0, sz)], acc.at[pl.ds(row, sz)])

    def _fill1(r, carry):
        for cc in range(D // 16):
            ones_v[r, pl.ds(cc * 16, 16)] = jnp.ones((16,), jnp.float32)
        return carry
    lax.fori_loop(0, _CC, _fill1, 0)
    plsc.subcore_barrier()

    def _body(j, carry):
        c0 = (wid + (2 * j) * _NW) * _CC
        c1 = (wid + (2 * j + 1) * _NW) * _CC
        h0 = pltpu.async_copy(dst_hbm.at[pl.ds(c0, _CC)], da, lsem)
        h1 = pltpu.async_copy(dst_hbm.at[pl.ds(c1, _CC)], db, lsem)
        h0.wait()
        s0 = pltpu.async_copy(ones_v, acc.at[da], ssem, add=True)
        h1.wait()
        s1 = pltpu.async_copy(ones_v, acc.at[db], ssem, add=True)
        s0.wait()
        s1.wait()
        return carry
    lax.fori_loop(0, _BODIES_C, _body, 0)

    @pl.when(wid < _EXTRA_C)
    def _():
        base = (_ITERS_C * _NW + wid) * _CC
        pltpu.sync_copy(dst_hbm.at[pl.ds(base, _CC)], da)
        pltpu.sync_copy(ones_v, acc.at[da], add=True)

    plsc.subcore_barrier()

    for row, sz in _acc_slices(base0, _CC):
        pltpu.sync_copy(acc.at[pl.ds(row, sz)], ones_v.at[pl.ds(0, sz)])
        pltpu.sync_copy(ones_v.at[pl.ds(0, sz)], cnt_out.at[cid, pl.ds(row, sz)])


@jax.jit
def _sc_scatter(nh, src, dst):
    mesh = plsc.VectorSubcoreMesh(core_axis_name="c", subcore_axis_name="s")
    sums = pl.kernel(
        _sc_sums_body,
        out_type=jax.ShapeDtypeStruct((_NC, _NP, D), jnp.float32),
        mesh=mesh,
        scratch_types=[
            pltpu.VMEM((_CS,), jnp.int32),
            pltpu.VMEM((_CS,), jnp.int32),
            pltpu.VMEM((_CS,), jnp.int32),
            pltpu.VMEM((_CS,), jnp.int32),
            pltpu.VMEM((_CS, D), jnp.float32),
            pltpu.VMEM((_CS, D), jnp.float32),
            pltpu.VMEM_SHARED((_NP, D), jnp.float32),
            pltpu.SemaphoreType.DMA,
            pltpu.SemaphoreType.DMA,
            pltpu.SemaphoreType.DMA,
            pltpu.SemaphoreType.DMA,
        ],
    )(nh, src, dst)
    cnts = pl.kernel(
        _sc_cnt_body,
        out_type=jax.ShapeDtypeStruct((_NC, _NP, D), jnp.float32),
        mesh=mesh,
        scratch_types=[
            pltpu.VMEM((_CC,), jnp.int32),
            pltpu.VMEM((_CC,), jnp.int32),
            pltpu.VMEM((_CC, D), jnp.float32),
            pltpu.VMEM_SHARED((_NP, D), jnp.float32),
            pltpu.SemaphoreType.DMA,
            pltpu.SemaphoreType.DMA,
        ],
    )(dst)
    return sums, cnts


def _mlp_body(sums_ref, cnt_ref, nh_ref, w1a_ref, w1b_ref, b1_ref,
              w2_ref, b2_ref, out_ref):
    s = sums_ref[0] + sums_ref[1]
    c = cnt_ref[0] + cnt_ref[1]
    cnt = jnp.maximum(c[:, 0:1], 1.0)
    agg = s / cnt
    x = jnp.dot(nh_ref[...], w1a_ref[...], preferred_element_type=jnp.float32)
    x = x + jnp.dot(agg, w1b_ref[...], preferred_element_type=jnp.float32)
    h = jnp.maximum(x + b1_ref[...], 0.0)
    out_ref[...] = (jnp.dot(h, w2_ref[...], preferred_element_type=jnp.float32)
                    + b2_ref[...])


_BLK = 1000


@jax.jit
def _mlp(sums, cnts, nh, w1a, w1b, b1, w2, b2):
    grid = (N // _BLK,)
    return pl.pallas_call(
        _mlp_body,
        grid=grid,
        in_specs=[
            pl.BlockSpec((_NC, _BLK, D), lambda i: (0, i, 0)),
            pl.BlockSpec((_NC, _BLK, D), lambda i: (0, i, 0)),
            pl.BlockSpec((_BLK, D), lambda i: (i, 0)),
            pl.BlockSpec((D, D), lambda i: (0, 0)),
            pl.BlockSpec((D, D), lambda i: (0, 0)),
            pl.BlockSpec((1, D), lambda i: (0, 0)),
            pl.BlockSpec((D, D), lambda i: (0, 0)),
            pl.BlockSpec((1, D), lambda i: (0, 0)),
        ],
        out_specs=pl.BlockSpec((_BLK, D), lambda i: (i, 0)),
        out_shape=jax.ShapeDtypeStruct((N, D), jnp.float32),
    )(sums, cnts, nh, w1a, w1b, b1, w2, b2)


def kernel(nh, eh, edge_index, W1, b1, W2, b2):
    src = edge_index[0]
    dst = edge_index[1]
    sums, cnts = _sc_scatter(nh, src, dst)
    n_h = _mlp(sums, cnts, nh, W1[:D], W1[D:], b1.reshape(1, D),
               W2, b2.reshape(1, D))
    return (n_h, eh)


# speedup vs baseline: 6.7691x; 1.0127x over previous
import jax
import jax.numpy as jnp
from jax import lax
from jax.experimental import pallas as pl
from jax.experimental.pallas import tpu as pltpu
from jax.experimental.pallas import tpu_sc as plsc

N = 10000
E = 320000
D = 128

_NC = 2
_NS = 16
_NW = _NC * _NS
_NP = 10112
_ROWS_PER_TILE = _NP // _NS

_CS = 64
_NCH_S = E // _CS
_ITERS_S = _NCH_S // _NW
_BODIES_S = _ITERS_S // 2
_EXTRA_S = _NCH_S - _ITERS_S * _NW

_CC = 128
_NCH_C = E // _CC
_ITERS_C = _NCH_C // _NW
_BODIES_C = _ITERS_C // 2
_EXTRA_C = _NCH_C - _ITERS_C * _NW


def _acc_slices(base0, piece):
    off = 0
    while off < _ROWS_PER_TILE:
        sz = min(piece, _ROWS_PER_TILE - off)
        yield base0 + off, sz
        off += sz


def _sc_body(nh_hbm, src_hbm, dst_hbm, sums_out, cnt_out,
             ia, ib, da, db, dc, dd, rows_a, rows_b, acc,
             lsem, ssem, ga, gb):
    cid = lax.axis_index("c")
    sid = lax.axis_index("s")
    wid = cid * _NS + sid
    base0 = sid * _ROWS_PER_TILE

    def _fill(val):
        def _f(r, carry):
            for cc in range(D // 16):
                rows_a[r, pl.ds(cc * 16, 16)] = val
                rows_b[r, pl.ds(cc * 16, 16)] = val
            return carry
        lax.fori_loop(0, _CS, _f, 0)

    _fill(jnp.zeros((16,), jnp.float32))
    for row, sz in _acc_slices(base0, _CS):
        pltpu.sync_copy(rows_a.at[pl.ds(0, sz)], acc.at[pl.ds(row, sz)])
    _fill(jnp.ones((16,), jnp.float32))
    plsc.subcore_barrier()

    def _cnt_body(j, carry):
        c0 = (wid + (2 * j) * _NW) * _CC
        c1 = (wid + (2 * j + 1) * _NW) * _CC
        h0 = pltpu.async_copy(dst_hbm.at[pl.ds(c0, _CS)], da, lsem)
        h1 = pltpu.async_copy(dst_hbm.at[pl.ds(c0 + _CS, _CS)], db, lsem)
        h2 = pltpu.async_copy(dst_hbm.at[pl.ds(c1, _CS)], dc, lsem)
        h3 = pltpu.async_copy(dst_hbm.at[pl.ds(c1 + _CS, _CS)], dd, lsem)
        h0.wait()
        s0 = pltpu.async_copy(rows_a, acc.at[da], ssem, add=True)
        h1.wait()
        s1 = pltpu.async_copy(rows_b, acc.at[db], ssem, add=True)
        h2.wait()
        s2 = pltpu.async_copy(rows_a, acc.at[dc], ssem, add=True)
        h3.wait()
        s3 = pltpu.async_copy(rows_b, acc.at[dd], ssem, add=True)
        s0.wait()
        s1.wait()
        s2.wait()
        s3.wait()
        return carry
    lax.fori_loop(0, _BODIES_C, _cnt_body, 0)

    @pl.when(wid < _EXTRA_C)
    def _():
        base = (_ITERS_C * _NW + wid) * _CC
        pltpu.sync_copy(dst_hbm.at[pl.ds(base, _CS)], da)
        pltpu.sync_copy(dst_hbm.at[pl.ds(base + _CS, _CS)], db)
        pltpu.sync_copy(rows_a, acc.at[da], add=True)
        pltpu.sync_copy(rows_b, acc.at[db], add=True)

    plsc.subcore_barrier()
    for row, sz in _acc_slices(base0, _CS):
        pltpu.sync_copy(acc.at[pl.ds(row, sz)], rows_b.at[pl.ds(0, sz)])
        pltpu.sync_copy(rows_b.at[pl.ds(0, sz)], cnt_out.at[cid, pl.ds(row, sz)])
    plsc.subcore_barrier()

    def _zrows(r, carry):
        for cc in range(D // 16):
            rows_a[r, pl.ds(cc * 16, 16)] = jnp.zeros((16,), jnp.float32)
        return carry
    lax.fori_loop(0, _CS, _zrows, 0)
    for row, sz in _acc_slices(base0, _CS):
        pltpu.sync_copy(rows_a.at[pl.ds(0, sz)], acc.at[pl.ds(row, sz)])
    plsc.subcore_barrier()

    def _sum_body(j, carry):
        c0 = (wid + (2 * j) * _NW) * _CS
        c1 = (wid + (2 * j + 1) * _NW) * _CS
        hs = (pltpu.async_copy(src_hbm.at[pl.ds(c0, _CS)], ia, lsem),
              pltpu.async_copy(dst_hbm.at[pl.ds(c0, _CS)], da, lsem),
              pltpu.async_copy(src_hbm.at[pl.ds(c1, _CS)], ib, lsem),
              pltpu.async_copy(dst_hbm.at[pl.ds(c1, _CS)], db, lsem))
        for h in hs:
            h.wait()
        gh0 = pltpu.async_copy(nh_hbm.at[ia], rows_a, ga)
        gh1 = pltpu.async_copy(nh_hbm.at[ib], rows_b, gb)
        gh0.wait()
        s0 = pltpu.async_copy(rows_a, acc.at[da], ssem, add=True)
        gh1.wait()
        s1 = pltpu.async_copy(rows_b, acc.at[db], ssem, add=True)
        s0.wait()
        s1.wait()
        return carry
    lax.fori_loop(0, _BODIES_S, _sum_body, 0)

    @pl.when(wid < _EXTRA_S)
    def _():
        base = (_ITERS_S * _NW + wid) * _CS
        pltpu.sync_copy(src_hbm.at[pl.ds(base, _CS)], ia)
        pltpu.sync_copy(dst_hbm.at[pl.ds(base, _CS)], da)
        pltpu.async_copy(nh_hbm.at[ia], rows_a, ga).wait()
        pltpu.sync_copy(rows_a, acc.at[da], add=True)

    plsc.subcore_barrier()
    for row, sz in _acc_slices(base0, _CS):
        pltpu.sync_copy(acc.at[pl.ds(row, sz)], rows_a.at[pl.ds(0, sz)])
        pltpu.sync_copy(rows_a.at[pl.ds(0, sz)], sums_out.at[cid, pl.ds(row, sz)])


@jax.jit
def _sc_scatter(nh, src, dst):
    mesh = plsc.VectorSubcoreMesh(core_axis_name="c", subcore_axis_name="s")
    return pl.kernel(
        _sc_body,
        out_type=(
            jax.ShapeDtypeStruct((_NC, _NP, D), jnp.float32),
            jax.ShapeDtypeStruct((_NC, _NP, D), jnp.float32),
        ),
        mesh=mesh,
        scratch_types=[
            pltpu.VMEM((_CS,), jnp.int32),
            pltpu.VMEM((_CS,), jnp.int32),
            pltpu.VMEM((_CS,), jnp.int32),
            pltpu.VMEM((_CS,), jnp.int32),
            pltpu.VMEM((_CS,), jnp.int32),
            pltpu.VMEM((_CS,), jnp.int32),
            pltpu.VMEM((_CS, D), jnp.float32),
            pltpu.VMEM((_CS, D), jnp.float32),
            pltpu.VMEM_SHARED((_NP, D), jnp.float32),
            pltpu.SemaphoreType.DMA,
            pltpu.SemaphoreType.DMA,
            pltpu.SemaphoreType.DMA,
            pltpu.SemaphoreType.DMA,
        ],
    )(nh, src, dst)


def _mlp_body(sums_ref, cnt_ref, nh_ref, w1a_ref, w1b_ref, b1_ref,
              w2_ref, b2_ref, out_ref):
    s = sums_ref[0] + sums_ref[1]
    c = cnt_ref[0] + cnt_ref[1]
    cnt = jnp.maximum(c[:, 0:1], 1.0)
    agg = s / cnt
    x = jnp.dot(nh_ref[...], w1a_ref[...], preferred_element_type=jnp.float32)
    x = x + jnp.dot(agg, w1b_ref[...], preferred_element_type=jnp.float32)
    h = jnp.maximum(x + b1_ref[...], 0.0)
    out_ref[...] = (jnp.dot(h, w2_ref[...], preferred_element_type=jnp.float32)
                    + b2_ref[...])


_BLK = 1000


@jax.jit
def _mlp(sums, cnts, nh, w1a, w1b, b1, w2, b2):
    grid = (N // _BLK,)
    return pl.pallas_call(
        _mlp_body,
        grid=grid,
        in_specs=[
            pl.BlockSpec((_NC, _BLK, D), lambda i: (0, i, 0)),
            pl.BlockSpec((_NC, _BLK, D), lambda i: (0, i, 0)),
            pl.BlockSpec((_BLK, D), lambda i: (i, 0)),
            pl.BlockSpec((D, D), lambda i: (0, 0)),
            pl.BlockSpec((D, D), lambda i: (0, 0)),
            pl.BlockSpec((1, D), lambda i: (0, 0)),
            pl.BlockSpec((D, D), lambda i: (0, 0)),
            pl.BlockSpec((1, D), lambda i: (0, 0)),
        ],
        out_specs=pl.BlockSpec((_BLK, D), lambda i: (i, 0)),
        out_shape=jax.ShapeDtypeStruct((N, D), jnp.float32),
    )(sums, cnts, nh, w1a, w1b, b1, w2, b2)


def kernel(nh, eh, edge_index, W1, b1, W2, b2):
    src = edge_index[0]
    dst = edge_index[1]
    sums, cnts = _sc_scatter(nh, src, dst)
    n_h = _mlp(sums, cnts, nh, W1[:D], W1[D:], b1.reshape(1, D),
               W2, b2.reshape(1, D))
    return (n_h, eh)


# speedup vs baseline: 7.5267x; 1.1119x over previous
import jax
import jax.numpy as jnp
from jax import lax
from jax.experimental import pallas as pl
from jax.experimental.pallas import tpu as pltpu
from jax.experimental.pallas import tpu_sc as plsc

N = 10000
E = 320000
D = 128

_NC = 2
_NS = 16
_NW = _NC * _NS
_NP = 10112
_ROWS_PER_TILE = _NP // _NS

_CS = 64
_NCH_S = E // _CS
_ITERS_S = _NCH_S // _NW
_BODIES_S = _ITERS_S // 2
_EXTRA_S = _NCH_S - _ITERS_S * _NW

_CC = 128
_NCH_C = E // _CC
_ITERS_C = _NCH_C // _NW
_BODIES_C = _ITERS_C // 2
_EXTRA_C = _NCH_C - _ITERS_C * _NW


def _acc_slices(base0, piece):
    off = 0
    while off < _ROWS_PER_TILE:
        sz = min(piece, _ROWS_PER_TILE - off)
        yield base0 + off, sz
        off += sz


def _sc_body(nh_hbm, src_hbm, dst_hbm, sums_out, cnt_out,
             ia, ib, da, db, dc, dd, rows_a, rows_b, acc,
             lsem, ssem, ga, gb):
    cid = lax.axis_index("c")
    sid = lax.axis_index("s")
    wid = cid * _NS + sid
    base0 = sid * _ROWS_PER_TILE

    def _fill(val):
        def _f(r, carry):
            for cc in range(D // 16):
                rows_a[r, pl.ds(cc * 16, 16)] = val
                rows_b[r, pl.ds(cc * 16, 16)] = val
            return carry
        lax.fori_loop(0, _CS, _f, 0)

    _fill(jnp.zeros((16,), jnp.float32))
    for row, sz in _acc_slices(base0, _CS):
        pltpu.sync_copy(rows_a.at[pl.ds(0, sz)], acc.at[pl.ds(row, sz)])
    _fill(jnp.ones((16,), jnp.float32))
    plsc.subcore_barrier()

    def _drain(n):
        for _ in range(n):
            pltpu.make_async_copy(nh_hbm.at[pl.ds(0, _CS)], rows_a, ssem).wait()

    def _cnt_fire(j):
        c0 = (wid + (2 * j) * _NW) * _CC
        c1 = (wid + (2 * j + 1) * _NW) * _CC
        h0 = pltpu.async_copy(dst_hbm.at[pl.ds(c0, _CS)], da, lsem)
        h1 = pltpu.async_copy(dst_hbm.at[pl.ds(c0 + _CS, _CS)], db, lsem)
        h2 = pltpu.async_copy(dst_hbm.at[pl.ds(c1, _CS)], dc, lsem)
        h3 = pltpu.async_copy(dst_hbm.at[pl.ds(c1 + _CS, _CS)], dd, lsem)
        h0.wait()
        pltpu.async_copy(rows_a, acc.at[da], ssem, add=True)
        h1.wait()
        pltpu.async_copy(rows_b, acc.at[db], ssem, add=True)
        h2.wait()
        pltpu.async_copy(rows_a, acc.at[dc], ssem, add=True)
        h3.wait()
        pltpu.async_copy(rows_b, acc.at[dd], ssem, add=True)

    _cnt_fire(0)

    def _cnt_body(j, carry):
        _drain(4)
        _cnt_fire(j)
        return carry
    lax.fori_loop(1, _BODIES_C, _cnt_body, 0)
    _drain(4)

    @pl.when(wid < _EXTRA_C)
    def _():
        base = (_ITERS_C * _NW + wid) * _CC
        pltpu.sync_copy(dst_hbm.at[pl.ds(base, _CS)], da)
        pltpu.sync_copy(dst_hbm.at[pl.ds(base + _CS, _CS)], db)
        pltpu.sync_copy(rows_a, acc.at[da], add=True)
        pltpu.sync_copy(rows_b, acc.at[db], add=True)

    plsc.subcore_barrier()
    for row, sz in _acc_slices(base0, _CS):
        pltpu.sync_copy(acc.at[pl.ds(row, sz)], rows_b.at[pl.ds(0, sz)])
        pltpu.sync_copy(rows_b.at[pl.ds(0, sz)], cnt_out.at[cid, pl.ds(row, sz)])
    plsc.subcore_barrier()

    def _zrows(r, carry):
        for cc in range(D // 16):
            rows_a[r, pl.ds(cc * 16, 16)] = jnp.zeros((16,), jnp.float32)
        return carry
    lax.fori_loop(0, _CS, _zrows, 0)
    for row, sz in _acc_slices(base0, _CS):
        pltpu.sync_copy(rows_a.at[pl.ds(0, sz)], acc.at[pl.ds(row, sz)])
    plsc.subcore_barrier()

    def _sum_half(q, d0, d1, drain_first):
        c0 = (wid + q * _NW) * _CS
        c1 = (wid + (q + 1) * _NW) * _CS
        hs = (pltpu.async_copy(src_hbm.at[pl.ds(c0, _CS)], ia, lsem),
              pltpu.async_copy(dst_hbm.at[pl.ds(c0, _CS)], d0, lsem),
              pltpu.async_copy(src_hbm.at[pl.ds(c1, _CS)], ib, lsem),
              pltpu.async_copy(dst_hbm.at[pl.ds(c1, _CS)], d1, lsem))
        for h in hs:
            h.wait()
        if drain_first:
            _drain(2)
        gh0 = pltpu.async_copy(nh_hbm.at[ia], rows_a, ga)
        gh1 = pltpu.async_copy(nh_hbm.at[ib], rows_b, gb)
        gh0.wait()
        pltpu.async_copy(rows_a, acc.at[d0], ssem, add=True)
        gh1.wait()
        pltpu.async_copy(rows_b, acc.at[d1], ssem, add=True)

    def _sum_super(k, drain_first):
        _sum_half(4 * k, da, db, drain_first)
        _sum_half(4 * k + 2, dc, dd, True)

    _sum_super(0, False)

    def _sum_body(k, carry):
        _sum_super(k, True)
        return carry
    lax.fori_loop(1, _BODIES_S // 2, _sum_body, 0)
    _drain(2)

    @pl.when(wid < _EXTRA_S)
    def _():
        base = (_ITERS_S * _NW + wid) * _CS
        pltpu.sync_copy(src_hbm.at[pl.ds(base, _CS)], ia)
        pltpu.sync_copy(dst_hbm.at[pl.ds(base, _CS)], da)
        pltpu.async_copy(nh_hbm.at[ia], rows_a, ga).wait()
        pltpu.sync_copy(rows_a, acc.at[da], add=True)

    plsc.subcore_barrier()
    for row, sz in _acc_slices(base0, _CS):
        pltpu.sync_copy(acc.at[pl.ds(row, sz)], rows_a.at[pl.ds(0, sz)])
        pltpu.sync_copy(rows_a.at[pl.ds(0, sz)], sums_out.at[cid, pl.ds(row, sz)])


@jax.jit
def _sc_scatter(nh, src, dst):
    mesh = plsc.VectorSubcoreMesh(core_axis_name="c", subcore_axis_name="s")
    return pl.kernel(
        _sc_body,
        out_type=(
            jax.ShapeDtypeStruct((_NC, _NP, D), jnp.float32),
            jax.ShapeDtypeStruct((_NC, _NP, D), jnp.float32),
        ),
        mesh=mesh,
        scratch_types=[
            pltpu.VMEM((_CS,), jnp.int32),
            pltpu.VMEM((_CS,), jnp.int32),
            pltpu.VMEM((_CS,), jnp.int32),
            pltpu.VMEM((_CS,), jnp.int32),
            pltpu.VMEM((_CS,), jnp.int32),
            pltpu.VMEM((_CS,), jnp.int32),
            pltpu.VMEM((_CS, D), jnp.float32),
            pltpu.VMEM((_CS, D), jnp.float32),
            pltpu.VMEM_SHARED((_NP, D), jnp.float32),
            pltpu.SemaphoreType.DMA,
            pltpu.SemaphoreType.DMA,
            pltpu.SemaphoreType.DMA,
            pltpu.SemaphoreType.DMA,
        ],
    )(nh, src, dst)


def _mlp_body(sums_ref, cnt_ref, nh_ref, w1a_ref, w1b_ref, b1_ref,
              w2_ref, b2_ref, out_ref):
    s = sums_ref[0] + sums_ref[1]
    c = cnt_ref[0] + cnt_ref[1]
    cnt = jnp.maximum(c[:, 0:1], 1.0)
    agg = s / cnt
    x = jnp.dot(nh_ref[...], w1a_ref[...], preferred_element_type=jnp.float32)
    x = x + jnp.dot(agg, w1b_ref[...], preferred_element_type=jnp.float32)
    h = jnp.maximum(x + b1_ref[...], 0.0)
    out_ref[...] = (jnp.dot(h, w2_ref[...], preferred_element_type=jnp.float32)
                    + b2_ref[...])


_BLK = 1000


@jax.jit
def _mlp(sums, cnts, nh, w1a, w1b, b1, w2, b2):
    grid = (N // _BLK,)
    return pl.pallas_call(
        _mlp_body,
        grid=grid,
        in_specs=[
            pl.BlockSpec((_NC, _BLK, D), lambda i: (0, i, 0)),
            pl.BlockSpec((_NC, _BLK, D), lambda i: (0, i, 0)),
            pl.BlockSpec((_BLK, D), lambda i: (i, 0)),
            pl.BlockSpec((D, D), lambda i: (0, 0)),
            pl.BlockSpec((D, D), lambda i: (0, 0)),
            pl.BlockSpec((1, D), lambda i: (0, 0)),
            pl.BlockSpec((D, D), lambda i: (0, 0)),
            pl.BlockSpec((1, D), lambda i: (0, 0)),
        ],
        out_specs=pl.BlockSpec((_BLK, D), lambda i: (i, 0)),
        out_shape=jax.ShapeDtypeStruct((N, D), jnp.float32),
    )(sums, cnts, nh, w1a, w1b, b1, w2, b2)


def kernel(nh, eh, edge_index, W1, b1, W2, b2):
    src = edge_index[0]
    dst = edge_index[1]
    sums, cnts = _sc_scatter(nh, src, dst)
    n_h = _mlp(sums, cnts, nh, W1[:D], W1[D:], b1.reshape(1, D),
               W2, b2.reshape(1, D))
    return (n_h, eh)
